# baseline (speedup 1.0000x reference)
import functools

import jax
import jax.numpy as jnp
import numpy as np
from jax import lax
from jax.experimental import pallas as pl
from jax.experimental.pallas import tpu as pltpu
from jax.experimental.pallas import tpu_sc as plsc

NMEM = 16
CHUNK = 512


def _mm(a, b, dims):
    return lax.dot_general(a, b, (dims, ((), ())),
                           precision=jax.lax.Precision.DEFAULT,
                           preferred_element_type=jnp.float32)


def _mmb(a, b, dims):
    return lax.dot_general(a.astype(jnp.bfloat16), b.astype(jnp.bfloat16),
                           (dims, ((), ())),
                           precision=jax.lax.Precision.DEFAULT,
                           preferred_element_type=jnp.float32)


def _mom_body(xe_ref, wq_ref, wk_ref, wv_ref, wr_ref,
              bq_ref, bk_ref, bv_ref, br_ref, rw_ref, wo_ref, bo_ref,
              out_ref, m_ref, wp_s, bp_s):
    i = pl.program_id(0)
    C = xe_ref.shape[0]
    HID = wo_ref.shape[0]
    NM1 = NMEM + 1

    @pl.when(i == 0)
    def _init():
        m_ref[...] = jnp.zeros_like(m_ref)
        wp_s[:, 0:HID] = wq_ref[...]
        wp_s[:, HID:2 * HID] = wk_ref[...]
        wp_s[:, 2 * HID:3 * HID] = wv_ref[...]
        bp_s[:, 0:HID] = bq_ref[...]
        bp_s[:, HID:2 * HID] = bk_ref[...]
        bp_s[:, 2 * HID:3 * HID] = bv_ref[...]

    xe = xe_ref[...]
    qkv = _mm(xe, wp_s[...], ((1,), (0,))) + bp_s[...]
    q = qkv[:, 0:HID]
    k = qkv[:, HID:2 * HID]
    v = qkv[:, 2 * HID:3 * HID]
    lt = _mm(wr_ref[...], xe, ((0,), (1,))) + br_ref[...]

    mx = jnp.max(lt, axis=0, keepdims=True)
    e = jnp.exp(lt - mx)
    p = e / jnp.sum(e, axis=0, keepdims=True)

    j = lax.broadcasted_iota(jnp.int32, (NMEM, C), 0)
    v1 = jnp.max(p, axis=0, keepdims=True)
    i1 = jnp.min(jnp.where(p == v1, j, NMEM), axis=0, keepdims=True)
    m1 = j == i1
    p2 = jnp.where(m1, -1.0, p)
    v2 = jnp.max(p2, axis=0, keepdims=True)
    i2 = jnp.min(jnp.where(p2 == v2, j, NMEM), axis=0, keepdims=True)
    m2 = j == i2
    wt = (jnp.where(m1, v1, 0.0) + jnp.where(m2, v2, 0.0)) / (v1 + v2)

    wt_full = jnp.concatenate([wt, jnp.ones((1, C), jnp.float32)], axis=0)
    w_rep = _mm(wt_full, rw_ref[...], ((0,), (0,)))

    q_tile = pltpu.repeat(q, NM1, axis=1)
    o = _mmb(w_rep * q_tile, m_ref[...], ((1,), (0,)))

    A = _mmb(q, k, ((1,), (1,)))
    G = _mm(wt, wt, ((0,), (0,))) + 1.0
    ti = lax.broadcasted_iota(jnp.int32, (C, C), 0)
    si = lax.broadcasted_iota(jnp.int32, (C, C), 1)
    P = jnp.where(ti >= si, A * G, 0.0)
    o = o + _mmb(P, v, ((1,), (0,)))

    res = _mmb(o, wo_ref[...], ((1,), (0,))) + bo_ref[...]
    out_ref[...] = res.reshape(C, 1, res.shape[1])

    k_tile = pltpu.repeat(k, NM1, axis=1)
    m_ref[...] += _mmb(w_rep * k_tile, v, ((0,), (0,)))


def _mom_dense(xe, Wq, Wk, Wv, Wr, bq, bk, bv, br, Rw, Wo, bo,
               interpret=False):
    S, EMB = xe.shape
    HID = Wo.shape[0]
    OUT = Wo.shape[1]
    NM1 = NMEM + 1
    C = CHUNK
    full = lambda shape: pl.BlockSpec(shape, lambda i: (0, 0))
    return pl.pallas_call(
        _mom_body,
        grid=(S // C,),
        in_specs=[
            pl.BlockSpec((C, EMB), lambda i: (i, 0)),
            full(Wq.shape), full(Wk.shape), full(Wv.shape), full(Wr.shape),
            full(bq.shape), full(bk.shape), full(bv.shape), full(br.shape),
            full(Rw.shape),
            full((HID, OUT)), full((1, OUT)),
        ],
        out_specs=pl.BlockSpec((C, 1, OUT), lambda i: (i, 0, 0)),
        out_shape=jax.ShapeDtypeStruct((S, 1, OUT), jnp.float32),
        scratch_shapes=[
            pltpu.VMEM((NM1 * HID, HID), jnp.float32),
            pltpu.VMEM((EMB, 3 * HID), jnp.float32),
            pltpu.VMEM((1, 3 * HID), jnp.float32),
        ],
        compiler_params=pltpu.CompilerParams(
            dimension_semantics=("arbitrary",)),
        interpret=interpret,
    )(xe, Wq, Wk, Wv, Wr, bq, bk, bv, br, Rw, Wo, bo)


def _sc_gather(table, idx):
    V, D = table.shape
    (B,) = idx.shape
    info = plsc.get_sparse_core_info()
    NC, NS = info.num_cores, info.num_subcores
    NW = NC * NS
    b_per_w = B // NW
    mesh = plsc.VectorSubcoreMesh(core_axis_name="c", subcore_axis_name="s")

    @functools.partial(
        pl.kernel, mesh=mesh,
        out_type=jax.ShapeDtypeStruct((B, D), jnp.float32),
        scratch_types=[
            pltpu.VMEM((b_per_w,), jnp.int32),
            pltpu.VMEM((b_per_w, D), jnp.float32),
            pltpu.SemaphoreType.DMA,
        ],
    )
    def gather_k(table_hbm, idx_hbm, out_hbm, idx_v, rows_v, sem):
        wid = lax.axis_index("s") * NC + lax.axis_index("c")
        base = wid * b_per_w
        pltpu.sync_copy(idx_hbm.at[pl.ds(base, b_per_w)], idx_v)
        pltpu.async_copy(table_hbm.at[idx_v], rows_v, sem).wait()
        pltpu.sync_copy(rows_v, out_hbm.at[pl.ds(base, b_per_w)])

    return gather_k(table, idx)


def kernel(x, emb, Wq, bq, Wk, bk, Wv, bv, Wr, br, Wo, bo):
    S, B = x.shape
    HID = Wq.shape[1]
    OUT = Wo.shape[1]
    NM1 = NMEM + 1
    idx = x.reshape(-1).astype(jnp.int32)
    xe = _sc_gather(emb, idx)
    Rw = jnp.asarray(np.kron(np.eye(NM1, dtype=np.float32),
                             np.ones((1, HID), np.float32)))
    return _mom_dense(xe, Wq, Wk, Wv, Wr,
                      bq.reshape(1, HID), bk.reshape(1, HID),
                      bv.reshape(1, HID), br.reshape(NMEM, 1),
                      Rw, Wo, bo.reshape(1, OUT))


# speedup vs baseline: 1.0133x; 1.0133x over previous
import functools

import jax
import jax.numpy as jnp
import numpy as np
from jax import lax
from jax.experimental import pallas as pl
from jax.experimental.pallas import tpu as pltpu
from jax.experimental.pallas import tpu_sc as plsc

NMEM = 16
CHUNK = 512


def _mm(a, b, dims):
    return lax.dot_general(a, b, (dims, ((), ())),
                           precision=jax.lax.Precision.DEFAULT,
                           preferred_element_type=jnp.float32)


def _mom_body(xe_ref, wq_ref, wk_ref, wv_ref, wr_ref,
              bq_ref, bk_ref, bv_ref, br_ref, rw_ref, wo_ref, bo_ref,
              out_ref, m_ref, wp_s, bp_s):
    i = pl.program_id(0)
    C = xe_ref.shape[0]
    HID = wo_ref.shape[0]
    NM1 = NMEM + 1

    @pl.when(i == 0)
    def _init():
        m_ref[...] = jnp.zeros_like(m_ref)
        wp_s[:, 0:HID] = wq_ref[...]
        wp_s[:, HID:2 * HID] = wk_ref[...]
        wp_s[:, 2 * HID:3 * HID] = wv_ref[...]
        bp_s[:, 0:HID] = bq_ref[...]
        bp_s[:, HID:2 * HID] = bk_ref[...]
        bp_s[:, 2 * HID:3 * HID] = bv_ref[...]

    xe = xe_ref[...]
    qkv = _mm(xe, wp_s[...], ((1,), (0,))) + bp_s[...]
    q = qkv[:, 0:HID]
    k = qkv[:, HID:2 * HID]
    v = qkv[:, 2 * HID:3 * HID]
    lt = _mm(wr_ref[...], xe, ((0,), (1,))) + br_ref[...]

    mx = jnp.max(lt, axis=0, keepdims=True)
    e = jnp.exp(lt - mx)
    p = e / jnp.sum(e, axis=0, keepdims=True)

    j = lax.broadcasted_iota(jnp.int32, (NMEM, C), 0)
    v1 = jnp.max(p, axis=0, keepdims=True)
    i1 = jnp.min(jnp.where(p == v1, j, NMEM), axis=0, keepdims=True)
    m1 = j == i1
    p2 = jnp.where(m1, -1.0, p)
    v2 = jnp.max(p2, axis=0, keepdims=True)
    i2 = jnp.min(jnp.where(p2 == v2, j, NMEM), axis=0, keepdims=True)
    m2 = j == i2
    wt = (jnp.where(m1, v1, 0.0) + jnp.where(m2, v2, 0.0)) / (v1 + v2)

    wt_full = jnp.concatenate([wt, jnp.ones((1, C), jnp.float32)], axis=0)
    w_rep = _mm(wt_full, rw_ref[...], ((0,), (0,)))

    q_tile = pltpu.repeat(q, NM1, axis=1)
    o = _mm(w_rep * q_tile, m_ref[...], ((1,), (0,)))

    A = _mm(q, k, ((1,), (1,)))
    G = _mm(wt, wt, ((0,), (0,))) + 1.0
    ti = lax.broadcasted_iota(jnp.int32, (C, C), 0)
    si = lax.broadcasted_iota(jnp.int32, (C, C), 1)
    P = jnp.where(ti >= si, A * G, 0.0)
    o = o + _mm(P, v, ((1,), (0,)))

    res = _mm(o, wo_ref[...], ((1,), (0,))) + bo_ref[...]
    out_ref[...] = res.reshape(C, 1, res.shape[1])

    k_tile = pltpu.repeat(k, NM1, axis=1)
    m_ref[...] += _mm(w_rep * k_tile, v, ((0,), (0,)))


def _mom_dense(xe, Wq, Wk, Wv, Wr, bq, bk, bv, br, Rw, Wo, bo,
               interpret=False):
    S, EMB = xe.shape
    HID = Wo.shape[0]
    OUT = Wo.shape[1]
    NM1 = NMEM + 1
    C = CHUNK
    full = lambda shape: pl.BlockSpec(shape, lambda i: (0, 0))
    return pl.pallas_call(
        _mom_body,
        grid=(S // C,),
        in_specs=[
            pl.BlockSpec((C, EMB), lambda i: (i, 0)),
            full(Wq.shape), full(Wk.shape), full(Wv.shape), full(Wr.shape),
            full(bq.shape), full(bk.shape), full(bv.shape), full(br.shape),
            full(Rw.shape),
            full((HID, OUT)), full((1, OUT)),
        ],
        out_specs=pl.BlockSpec((C, 1, OUT), lambda i: (i, 0, 0)),
        out_shape=jax.ShapeDtypeStruct((S, 1, OUT), jnp.float32),
        scratch_shapes=[
            pltpu.VMEM((NM1 * HID, HID), jnp.float32),
            pltpu.VMEM((EMB, 3 * HID), jnp.float32),
            pltpu.VMEM((1, 3 * HID), jnp.float32),
        ],
        compiler_params=pltpu.CompilerParams(
            dimension_semantics=("arbitrary",)),
        interpret=interpret,
    )(xe, Wq, Wk, Wv, Wr, bq, bk, bv, br, Rw, Wo, bo)


def _sc_gather(table, idx):
    V, D = table.shape
    (B,) = idx.shape
    info = plsc.get_sparse_core_info()
    NC, NS = info.num_cores, info.num_subcores
    NW = NC * NS
    b_per_w = B // NW
    mesh = plsc.VectorSubcoreMesh(core_axis_name="c", subcore_axis_name="s")

    @functools.partial(
        pl.kernel, mesh=mesh,
        out_type=jax.ShapeDtypeStruct((B, D), jnp.float32),
        scratch_types=[
            pltpu.VMEM((b_per_w,), jnp.int32),
            pltpu.VMEM((b_per_w, D), jnp.float32),
            pltpu.SemaphoreType.DMA,
        ],
    )
    def gather_k(table_hbm, idx_hbm, out_hbm, idx_v, rows_v, sem):
        wid = lax.axis_index("s") * NC + lax.axis_index("c")
        base = wid * b_per_w
        pltpu.sync_copy(idx_hbm.at[pl.ds(base, b_per_w)], idx_v)
        pltpu.async_copy(table_hbm.at[idx_v], rows_v, sem).wait()
        pltpu.sync_copy(rows_v, out_hbm.at[pl.ds(base, b_per_w)])

    return gather_k(table, idx)


def kernel(x, emb, Wq, bq, Wk, bk, Wv, bv, Wr, br, Wo, bo):
    S, B = x.shape
    HID = Wq.shape[1]
    OUT = Wo.shape[1]
    NM1 = NMEM + 1
    idx = x.reshape(-1).astype(jnp.int32)
    xe = _sc_gather(emb, idx)
    Rw = jnp.asarray(np.kron(np.eye(NM1, dtype=np.float32),
                             np.ones((1, HID), np.float32)))
    return _mom_dense(xe, Wq, Wk, Wv, Wr,
                      bq.reshape(1, HID), bk.reshape(1, HID),
                      bv.reshape(1, HID), br.reshape(NMEM, 1),
                      Rw, Wo, bo.reshape(1, OUT))


# speedup vs baseline: 1.0168x; 1.0035x over previous
import functools

import jax
import jax.numpy as jnp
import numpy as np
from jax import lax
from jax.experimental import pallas as pl
from jax.experimental.pallas import tpu as pltpu
from jax.experimental.pallas import tpu_sc as plsc

NMEM = 16
CHUNK = 512


def _mm(a, b, dims):
    return lax.dot_general(a, b, (dims, ((), ())),
                           precision=jax.lax.Precision.DEFAULT,
                           preferred_element_type=jnp.float32)


def _mom_body(xe_ref, wq_ref, wk_ref, wv_ref, wr_ref,
              bq_ref, bk_ref, bv_ref, br_ref, rw_ref, wo_ref, bo_ref,
              out_ref, m_ref, wp_s, bp_s):
    i = pl.program_id(0)
    C = xe_ref.shape[0]
    HID = wo_ref.shape[0]
    NM1 = NMEM + 1

    @pl.when(i == 0)
    def _init():
        m_ref[...] = jnp.zeros_like(m_ref)
        wp_s[:, 0:HID] = wq_ref[...]
        wp_s[:, HID:2 * HID] = wk_ref[...]
        wp_s[:, 2 * HID:3 * HID] = wv_ref[...]
        bp_s[:, 0:HID] = bq_ref[...]
        bp_s[:, HID:2 * HID] = bk_ref[...]
        bp_s[:, 2 * HID:3 * HID] = bv_ref[...]

    xe = xe_ref[...]
    qkv = _mm(xe, wp_s[...], ((1,), (0,))) + bp_s[...]
    q = qkv[:, 0:HID]
    k = qkv[:, HID:2 * HID]
    v = qkv[:, 2 * HID:3 * HID]
    lt = _mm(wr_ref[...], xe, ((0,), (1,))) + br_ref[...]

    mx = jnp.max(lt, axis=0, keepdims=True)
    e = jnp.exp(lt - mx)
    p = e / jnp.sum(e, axis=0, keepdims=True)

    j = lax.broadcasted_iota(jnp.int32, (NMEM, C), 0)
    v1 = jnp.max(p, axis=0, keepdims=True)
    i1 = jnp.min(jnp.where(p == v1, j, NMEM), axis=0, keepdims=True)
    m1 = j == i1
    p2 = jnp.where(m1, -1.0, p)
    v2 = jnp.max(p2, axis=0, keepdims=True)
    i2 = jnp.min(jnp.where(p2 == v2, j, NMEM), axis=0, keepdims=True)
    m2 = j == i2
    wt = (jnp.where(m1, v1, 0.0) + jnp.where(m2, v2, 0.0)) / (v1 + v2)

    wt_full = jnp.concatenate([wt, jnp.ones((1, C), jnp.float32)], axis=0)
    w_rep = _mm(wt_full, rw_ref[...], ((0,), (0,)))

    q_tile = pltpu.repeat(q, NM1, axis=1)
    o = _mm(w_rep * q_tile, m_ref[...], ((1,), (0,)))

    A = _mm(q, k, ((1,), (1,)))
    G = _mm(wt, wt, ((0,), (0,))) + 1.0
    ti = lax.broadcasted_iota(jnp.int32, (C, C), 0)
    si = lax.broadcasted_iota(jnp.int32, (C, C), 1)
    P = jnp.where(ti >= si, A * G, 0.0)
    o = o + _mm(P, v, ((1,), (0,)))

    res = _mm(o, wo_ref[...], ((1,), (0,))) + bo_ref[...]
    out_ref[...] = res.reshape(C, 1, res.shape[1])

    k_tile = pltpu.repeat(k, NM1, axis=1)
    m_ref[...] += _mm(w_rep * k_tile, v, ((0,), (0,)))


def _mom_dense(xe, Wq, Wk, Wv, Wr, bq, bk, bv, br, Rw, Wo, bo,
               interpret=False):
    S, EMB = xe.shape
    HID = Wo.shape[0]
    OUT = Wo.shape[1]
    NM1 = NMEM + 1
    C = CHUNK
    full = lambda shape: pl.BlockSpec(shape, lambda i: (0, 0))
    return pl.pallas_call(
        _mom_body,
        grid=(S // C,),
        in_specs=[
            pl.BlockSpec((C, EMB), lambda i: (i, 0)),
            full(Wq.shape), full(Wk.shape), full(Wv.shape), full(Wr.shape),
            full(bq.shape), full(bk.shape), full(bv.shape), full(br.shape),
            full(Rw.shape),
            full((HID, OUT)), full((1, OUT)),
        ],
        out_specs=pl.BlockSpec((C, 1, OUT), lambda i: (i, 0, 0)),
        out_shape=jax.ShapeDtypeStruct((S, 1, OUT), jnp.float32),
        scratch_shapes=[
            pltpu.VMEM((NM1 * HID, HID), jnp.float32),
            pltpu.VMEM((EMB, 3 * HID), jnp.float32),
            pltpu.VMEM((1, 3 * HID), jnp.float32),
        ],
        compiler_params=pltpu.CompilerParams(
            dimension_semantics=("arbitrary",)),
        interpret=interpret,
    )(xe, Wq, Wk, Wv, Wr, bq, bk, bv, br, Rw, Wo, bo)


def _sc_gather(table, idx):
    V, D = table.shape
    (B,) = idx.shape
    info = plsc.get_sparse_core_info()
    NC, NS = info.num_cores, info.num_subcores
    NW = NC * NS
    b_per_w = B // NW
    mesh = plsc.VectorSubcoreMesh(core_axis_name="c", subcore_axis_name="s")

    @functools.partial(
        pl.kernel, mesh=mesh,
        out_type=jax.ShapeDtypeStruct((B, D), jnp.float32),
        scratch_types=[
            pltpu.VMEM((b_per_w,), jnp.int32),
            pltpu.VMEM((b_per_w, D), jnp.float32),
            pltpu.SemaphoreType.DMA,
            pltpu.SemaphoreType.DMA,
            pltpu.SemaphoreType.DMA,
            pltpu.SemaphoreType.DMA,
        ],
    )
    def gather_k(table_hbm, idx_hbm, out_hbm, idx_v, rows_v,
                 sem_g0, sem_g1, sem_w0, sem_w1):
        wid = lax.axis_index("s") * NC + lax.axis_index("c")
        base = wid * b_per_w
        h = b_per_w // 2
        pltpu.sync_copy(idx_hbm.at[pl.ds(base, b_per_w)], idx_v)
        g0 = pltpu.async_copy(table_hbm.at[idx_v.at[pl.ds(0, h)]],
                              rows_v.at[pl.ds(0, h)], sem_g0)
        g1 = pltpu.async_copy(table_hbm.at[idx_v.at[pl.ds(h, h)]],
                              rows_v.at[pl.ds(h, h)], sem_g1)
        g0.wait()
        w0 = pltpu.async_copy(rows_v.at[pl.ds(0, h)],
                              out_hbm.at[pl.ds(base, h)], sem_w0)
        g1.wait()
        w1 = pltpu.async_copy(rows_v.at[pl.ds(h, h)],
                              out_hbm.at[pl.ds(base + h, h)], sem_w1)
        w0.wait()
        w1.wait()

    return gather_k(table, idx)


def kernel(x, emb, Wq, bq, Wk, bk, Wv, bv, Wr, br, Wo, bo):
    S, B = x.shape
    HID = Wq.shape[1]
    OUT = Wo.shape[1]
    NM1 = NMEM + 1
    idx = x.reshape(-1).astype(jnp.int32)
    xe = _sc_gather(emb, idx)
    Rw = jnp.asarray(np.kron(np.eye(NM1, dtype=np.float32),
                             np.ones((1, HID), np.float32)))
    return _mom_dense(xe, Wq, Wk, Wv, Wr,
                      bq.reshape(1, HID), bk.reshape(1, HID),
                      bv.reshape(1, HID), br.reshape(NMEM, 1),
                      Rw, Wo, bo.reshape(1, OUT))
